# Initial kernel scaffold; baseline (speedup 1.0000x reference)
import jax
import jax.numpy as jnp
from jax.experimental import pallas as pl


def kernel(input_image, guide_image, output_image, weight_image):
    raise NotImplementedError("write your pallas kernel here")


# speedup vs baseline: 1339.2533x; 1339.2533x over previous
import functools
import numpy as np

import jax
import jax.numpy as jnp
from jax.experimental import pallas as pl

_GH = 24
_GW = 24
_GD = 16
_REG_LAMBDA = 1e-7

_S_PAIRS = [(i, j) for i in range(4) for j in range(i, 4)]
_T_PAIRS = [(k, i) for k in range(3) for i in range(4)]
_NJ = len(_S_PAIRS) + len(_T_PAIRS)


def _spatial_weights(g, n):
    pos = (np.arange(n, dtype=np.float64) + 0.5) * (g - 1) / n
    i0 = np.clip(np.floor(pos).astype(np.int64), 0, g - 1)
    i1 = np.minimum(i0 + 1, g - 1)
    w = (pos - i0).astype(np.float32)
    m = np.zeros((g, n), dtype=np.float32)
    m[i0, np.arange(n)] += 1.0 - w
    m[i1, np.arange(n)] += w
    return m


def _splat_body(inp_ref, guide_ref, outp_ref, wgt_ref, wy_ref, wxt_ref, g_ref):
    f32 = jnp.float32
    a0 = inp_ref[0, 0]
    a1 = inp_ref[0, 1]
    a2 = inp_ref[0, 2]
    ones = jnp.ones_like(a0)
    wmean = (wgt_ref[0, 0] + wgt_ref[0, 1] + wgt_ref[0, 2]) * (1.0 / 3.0)
    gz = guide_ref[0, 0] * (_GD - 1)
    z0 = jnp.clip(jnp.floor(gz).astype(jnp.int32), 0, _GD - 1)
    z1 = jnp.minimum(z0 + 1, _GD - 1)
    wz = gz - z0.astype(f32)
    a = (a0, a1, a2, ones)
    o = (outp_ref[0, 0], outp_ref[0, 1], outp_ref[0, 2])
    wa = tuple(x * wmean for x in a)
    wy = wy_ref[...]
    wxt = wxt_ref[...]

    hi = jax.lax.Precision.HIGHEST

    def zeta_step(zeta, _):
        uz = jnp.where(z0 == zeta, 1.0 - wz, 0.0) + jnp.where(z1 == zeta, wz, 0.0)
        for j, (i1, i2) in enumerate(_S_PAIRS):
            p = (wa[i1] * a[i2]) * uz
            g_ref[zeta, j] = jnp.dot(
                jnp.dot(wy, p, preferred_element_type=f32, precision=hi),
                wxt, preferred_element_type=f32, precision=hi)
        for j, (k, i1) in enumerate(_T_PAIRS):
            p = (o[k] * wa[i1]) * uz
            g_ref[zeta, len(_S_PAIRS) + j] = jnp.dot(
                jnp.dot(wy, p, preferred_element_type=f32, precision=hi),
                wxt, preferred_element_type=f32, precision=hi)
        return 0

    jax.lax.fori_loop(0, _GD, zeta_step, 0, unroll=False)


def _inv4_sym(m):
    s0 = m[0, 0] * m[1, 1] - m[1, 0] * m[0, 1]
    s1 = m[0, 0] * m[1, 2] - m[1, 0] * m[0, 2]
    s2 = m[0, 0] * m[1, 3] - m[1, 0] * m[0, 3]
    s3 = m[0, 1] * m[1, 2] - m[1, 1] * m[0, 2]
    s4 = m[0, 1] * m[1, 3] - m[1, 1] * m[0, 3]
    s5 = m[0, 2] * m[1, 3] - m[1, 2] * m[0, 3]
    c5 = m[2, 2] * m[3, 3] - m[3, 2] * m[2, 3]
    c4 = m[2, 1] * m[3, 3] - m[3, 1] * m[2, 3]
    c3 = m[2, 1] * m[3, 2] - m[3, 1] * m[2, 2]
    c2 = m[2, 0] * m[3, 3] - m[3, 0] * m[2, 3]
    c1 = m[2, 0] * m[3, 2] - m[3, 0] * m[2, 2]
    c0 = m[2, 0] * m[3, 1] - m[3, 0] * m[2, 1]
    det = s0 * c5 - s1 * c4 + s2 * c3 + s3 * c2 - s4 * c1 + s5 * c0
    rdet = 1.0 / det
    inv = {}
    inv[0, 0] = (m[1, 1] * c5 - m[1, 2] * c4 + m[1, 3] * c3) * rdet
    inv[0, 1] = (-m[0, 1] * c5 + m[0, 2] * c4 - m[0, 3] * c3) * rdet
    inv[0, 2] = (m[3, 1] * s5 - m[3, 2] * s4 + m[3, 3] * s3) * rdet
    inv[0, 3] = (-m[2, 1] * s5 + m[2, 2] * s4 - m[2, 3] * s3) * rdet
    inv[1, 0] = (-m[1, 0] * c5 + m[1, 2] * c2 - m[1, 3] * c1) * rdet
    inv[1, 1] = (m[0, 0] * c5 - m[0, 2] * c2 + m[0, 3] * c1) * rdet
    inv[1, 2] = (-m[3, 0] * s5 + m[3, 2] * s2 - m[3, 3] * s1) * rdet
    inv[1, 3] = (m[2, 0] * s5 - m[2, 2] * s2 + m[2, 3] * s1) * rdet
    inv[2, 0] = (m[1, 0] * c4 - m[1, 1] * c2 + m[1, 3] * c0) * rdet
    inv[2, 1] = (-m[0, 0] * c4 + m[0, 1] * c2 - m[0, 3] * c0) * rdet
    inv[2, 2] = (m[3, 0] * s4 - m[3, 1] * s2 + m[3, 3] * s0) * rdet
    inv[2, 3] = (-m[2, 0] * s4 + m[2, 1] * s2 - m[2, 3] * s0) * rdet
    inv[3, 0] = (-m[1, 0] * c3 + m[1, 1] * c1 - m[1, 2] * c0) * rdet
    inv[3, 1] = (m[0, 0] * c3 - m[0, 1] * c1 + m[0, 2] * c0) * rdet
    inv[3, 2] = (-m[3, 0] * s3 + m[3, 1] * s1 - m[3, 2] * s0) * rdet
    inv[3, 3] = (m[2, 0] * s3 - m[2, 1] * s1 + m[2, 2] * s0) * rdet
    return inv


def _solve_body(g_ref, out_ref):
    S = {}
    for j, (i1, i2) in enumerate(_S_PAIRS):
        S[i1, i2] = g_ref[:, j]
        S[i2, i1] = S[i1, i2]
    T = {}
    for j, (k, i1) in enumerate(_T_PAIRS):
        T[k, i1] = g_ref[:, len(_S_PAIRS) + j]

    counts = S[3, 3]
    wl = _REG_LAMBDA * (counts + 1.0)

    gcs = jnp.sum(counts)
    wlg = _REG_LAMBDA * (gcs + 1.0)
    gain_g = [jnp.sum(T[k, 3]) / (jnp.sum(S[k, 3]) + wlg) for k in range(3)]
    zero_mask = counts == 0.0
    mixed = [jnp.where(zero_mask, gain_g[k], T[k, 3] / (S[k, 3] + wl))
             for k in range(3)]

    Sr = {}
    for i in range(4):
        for j in range(4):
            Sr[i, j] = S[i, j] + wl if i == j else S[i, j]
    Tr = {}
    for k in range(3):
        for i in range(4):
            Tr[k, i] = T[k, i] + wl * mixed[k] if i == k else T[k, i]

    amax = Sr[0, 0]
    for i in range(4):
        for j in range(4):
            amax = jnp.maximum(amax, jnp.abs(Sr[i, j]))
    scale = 1.0 / amax
    Sn = {k: v * scale for k, v in Sr.items()}
    inv = _inv4_sym(Sn)
    for k in range(3):
        for i in range(4):
            acc = Tr[k, 0] * inv[0, i]
            for q in range(1, 4):
                acc = acc + Tr[k, q] * inv[q, i]
            out_ref[k, i] = acc * scale


@jax.jit
def kernel(input_image, guide_image, output_image, weight_image):
    B, C, H, W = input_image.shape
    dtype = input_image.dtype
    wy = jnp.asarray(_spatial_weights(_GH, H))
    wxt = jnp.asarray(_spatial_weights(_GW, W).T)

    g = pl.pallas_call(
        _splat_body,
        out_shape=jax.ShapeDtypeStruct((_GD, _NJ, _GH, _GW), dtype),
    )(input_image, guide_image, output_image, weight_image, wy, wxt)

    gamma = pl.pallas_call(
        _solve_body,
        out_shape=jax.ShapeDtypeStruct((3, 4, _GD, _GH, _GW), dtype),
    )(g)

    return jnp.transpose(gamma, (3, 4, 2, 0, 1))[None]


# speedup vs baseline: 2068.8833x; 1.5448x over previous
import functools
import numpy as np

import jax
import jax.numpy as jnp
from jax.experimental import pallas as pl

_GH = 24
_GW = 24
_GD = 16
_REG_LAMBDA = 1e-7

_S_PAIRS = [(i, j) for i in range(4) for j in range(i, 4)]
_T_PAIRS = [(k, i) for k in range(3) for i in range(4)]
_NJ = len(_S_PAIRS) + len(_T_PAIRS)


def _spatial_weights(g, n):
    pos = (np.arange(n, dtype=np.float64) + 0.5) * (g - 1) / n
    i0 = np.clip(np.floor(pos).astype(np.int64), 0, g - 1)
    i1 = np.minimum(i0 + 1, g - 1)
    w = (pos - i0).astype(np.float32)
    m = np.zeros((g, n), dtype=np.float32)
    m[i0, np.arange(n)] += 1.0 - w
    m[i1, np.arange(n)] += w
    return m


def _split_bf16(x):
    hi = x.astype(jnp.bfloat16)
    lo = (x - hi.astype(jnp.float32)).astype(jnp.bfloat16)
    return hi, lo


def _splat_body(inp_ref, guide_ref, outp_ref, wgt_ref, wyh_ref, wyl_ref,
                wxth_ref, wxtl_ref, g_ref):
    f32 = jnp.float32
    a0 = inp_ref[0, 0]
    a1 = inp_ref[0, 1]
    a2 = inp_ref[0, 2]
    ones = jnp.ones_like(a0)
    wmean = (wgt_ref[0, 0] + wgt_ref[0, 1] + wgt_ref[0, 2]) * (1.0 / 3.0)
    gz = guide_ref[0, 0] * (_GD - 1)
    z0 = jnp.clip(jnp.floor(gz).astype(jnp.int32), 0, _GD - 1)
    z1 = jnp.minimum(z0 + 1, _GD - 1)
    wz = gz - z0.astype(f32)
    a = (a0, a1, a2, ones)
    o = (outp_ref[0, 0], outp_ref[0, 1], outp_ref[0, 2])
    wa = tuple(x * wmean for x in a)
    wyh, wyl = wyh_ref[...], wyl_ref[...]
    wxth, wxtl = wxth_ref[...], wxtl_ref[...]

    v = ([wa[i1] * a[i2] for (i1, i2) in _S_PAIRS]
         + [o[k] * wa[i1] for (k, i1) in _T_PAIRS])

    def dot3(p):
        ph, plo = _split_bf16(p)
        return (jnp.dot(wyh, ph, preferred_element_type=f32)
                + jnp.dot(wyh, plo, preferred_element_type=f32)
                + jnp.dot(wyl, ph, preferred_element_type=f32))

    def dot3r(q):
        qh, ql = _split_bf16(q)
        return (jnp.dot(qh, wxth, preferred_element_type=f32)
                + jnp.dot(qh, wxtl, preferred_element_type=f32)
                + jnp.dot(ql, wxth, preferred_element_type=f32))

    def zeta_step(zeta, _):
        uz = jnp.where(z0 == zeta, 1.0 - wz, 0.0) + jnp.where(z1 == zeta, wz, 0.0)
        for j in range(_NJ):
            g_ref[zeta, j] = dot3r(dot3(v[j] * uz))
        return 0

    jax.lax.fori_loop(0, _GD, zeta_step, 0, unroll=False)


def _inv4_sym(m):
    s0 = m[0, 0] * m[1, 1] - m[1, 0] * m[0, 1]
    s1 = m[0, 0] * m[1, 2] - m[1, 0] * m[0, 2]
    s2 = m[0, 0] * m[1, 3] - m[1, 0] * m[0, 3]
    s3 = m[0, 1] * m[1, 2] - m[1, 1] * m[0, 2]
    s4 = m[0, 1] * m[1, 3] - m[1, 1] * m[0, 3]
    s5 = m[0, 2] * m[1, 3] - m[1, 2] * m[0, 3]
    c5 = m[2, 2] * m[3, 3] - m[3, 2] * m[2, 3]
    c4 = m[2, 1] * m[3, 3] - m[3, 1] * m[2, 3]
    c3 = m[2, 1] * m[3, 2] - m[3, 1] * m[2, 2]
    c2 = m[2, 0] * m[3, 3] - m[3, 0] * m[2, 3]
    c1 = m[2, 0] * m[3, 2] - m[3, 0] * m[2, 2]
    c0 = m[2, 0] * m[3, 1] - m[3, 0] * m[2, 1]
    det = s0 * c5 - s1 * c4 + s2 * c3 + s3 * c2 - s4 * c1 + s5 * c0
    rdet = 1.0 / det
    inv = {}
    inv[0, 0] = (m[1, 1] * c5 - m[1, 2] * c4 + m[1, 3] * c3) * rdet
    inv[0, 1] = (-m[0, 1] * c5 + m[0, 2] * c4 - m[0, 3] * c3) * rdet
    inv[0, 2] = (m[3, 1] * s5 - m[3, 2] * s4 + m[3, 3] * s3) * rdet
    inv[0, 3] = (-m[2, 1] * s5 + m[2, 2] * s4 - m[2, 3] * s3) * rdet
    inv[1, 0] = (-m[1, 0] * c5 + m[1, 2] * c2 - m[1, 3] * c1) * rdet
    inv[1, 1] = (m[0, 0] * c5 - m[0, 2] * c2 + m[0, 3] * c1) * rdet
    inv[1, 2] = (-m[3, 0] * s5 + m[3, 2] * s2 - m[3, 3] * s1) * rdet
    inv[1, 3] = (m[2, 0] * s5 - m[2, 2] * s2 + m[2, 3] * s1) * rdet
    inv[2, 0] = (m[1, 0] * c4 - m[1, 1] * c2 + m[1, 3] * c0) * rdet
    inv[2, 1] = (-m[0, 0] * c4 + m[0, 1] * c2 - m[0, 3] * c0) * rdet
    inv[2, 2] = (m[3, 0] * s4 - m[3, 1] * s2 + m[3, 3] * s0) * rdet
    inv[2, 3] = (-m[2, 0] * s4 + m[2, 1] * s2 - m[2, 3] * s0) * rdet
    inv[3, 0] = (-m[1, 0] * c3 + m[1, 1] * c1 - m[1, 2] * c0) * rdet
    inv[3, 1] = (m[0, 0] * c3 - m[0, 1] * c1 + m[0, 2] * c0) * rdet
    inv[3, 2] = (-m[3, 0] * s3 + m[3, 1] * s1 - m[3, 2] * s0) * rdet
    inv[3, 3] = (m[2, 0] * s3 - m[2, 1] * s1 + m[2, 2] * s0) * rdet
    return inv


def _solve_body(g_ref, out_ref):
    S = {}
    for j, (i1, i2) in enumerate(_S_PAIRS):
        S[i1, i2] = g_ref[:, j]
        S[i2, i1] = S[i1, i2]
    T = {}
    for j, (k, i1) in enumerate(_T_PAIRS):
        T[k, i1] = g_ref[:, len(_S_PAIRS) + j]

    counts = S[3, 3]
    wl = _REG_LAMBDA * (counts + 1.0)

    gcs = jnp.sum(counts)
    wlg = _REG_LAMBDA * (gcs + 1.0)
    gain_g = [jnp.sum(T[k, 3]) / (jnp.sum(S[k, 3]) + wlg) for k in range(3)]
    zero_mask = counts == 0.0
    mixed = [jnp.where(zero_mask, gain_g[k], T[k, 3] / (S[k, 3] + wl))
             for k in range(3)]

    Sr = {}
    for i in range(4):
        for j in range(4):
            Sr[i, j] = S[i, j] + wl if i == j else S[i, j]
    Tr = {}
    for k in range(3):
        for i in range(4):
            Tr[k, i] = T[k, i] + wl * mixed[k] if i == k else T[k, i]

    amax = Sr[0, 0]
    for i in range(4):
        for j in range(4):
            amax = jnp.maximum(amax, jnp.abs(Sr[i, j]))
    scale = 1.0 / amax
    Sn = {k: v * scale for k, v in Sr.items()}
    inv = _inv4_sym(Sn)
    for k in range(3):
        for i in range(4):
            acc = Tr[k, 0] * inv[0, i]
            for q in range(1, 4):
                acc = acc + Tr[k, q] * inv[q, i]
            out_ref[k, i] = acc * scale


@jax.jit
def kernel(input_image, guide_image, output_image, weight_image):
    B, C, H, W = input_image.shape
    dtype = input_image.dtype
    wy = jnp.asarray(_spatial_weights(_GH, H))
    wxt = jnp.asarray(_spatial_weights(_GW, W).T)
    wyh, wyl = _split_bf16(wy)
    wxth, wxtl = _split_bf16(wxt)

    g = pl.pallas_call(
        _splat_body,
        out_shape=jax.ShapeDtypeStruct((_GD, _NJ, _GH, _GW), dtype),
    )(input_image, guide_image, output_image, weight_image,
      wyh, wyl, wxth, wxtl)

    gamma = pl.pallas_call(
        _solve_body,
        out_shape=jax.ShapeDtypeStruct((3, 4, _GD, _GH, _GW), dtype),
    )(g)

    return jnp.transpose(gamma, (3, 4, 2, 0, 1))[None]


# speedup vs baseline: 2099.4087x; 1.0148x over previous
import functools
import numpy as np

import jax
import jax.numpy as jnp
from jax.experimental import pallas as pl

_GH = 24
_GW = 24
_GD = 16
_REG_LAMBDA = 1e-7

_S_PAIRS = [(i, j) for i in range(4) for j in range(i, 4)]
_T_PAIRS = [(k, i) for k in range(3) for i in range(4)]
_NJ = len(_S_PAIRS) + len(_T_PAIRS)


def _spatial_weights(g, n):
    pos = (np.arange(n, dtype=np.float64) + 0.5) * (g - 1) / n
    i0 = np.clip(np.floor(pos).astype(np.int64), 0, g - 1)
    i1 = np.minimum(i0 + 1, g - 1)
    w = (pos - i0).astype(np.float32)
    m = np.zeros((g, n), dtype=np.float32)
    m[i0, np.arange(n)] += 1.0 - w
    m[i1, np.arange(n)] += w
    return m


def _split_bf16(x):
    hi = x.astype(jnp.bfloat16)
    lo = (x - hi.astype(jnp.float32)).astype(jnp.bfloat16)
    return hi, lo


def _splat_body(inp_ref, guide_ref, outp_ref, wgt_ref, wyh_ref, wyl_ref,
                wxth_ref, wxtl_ref, g_ref):
    f32 = jnp.float32
    a0 = inp_ref[0, 0]
    a1 = inp_ref[0, 1]
    a2 = inp_ref[0, 2]
    ones = jnp.ones_like(a0)
    wmean = (wgt_ref[0, 0] + wgt_ref[0, 1] + wgt_ref[0, 2]) * (1.0 / 3.0)
    gz = guide_ref[0, 0] * (_GD - 1)
    z0 = jnp.clip(jnp.floor(gz).astype(jnp.int32), 0, _GD - 1)
    z1 = jnp.minimum(z0 + 1, _GD - 1)
    wz = gz - z0.astype(f32)
    a = (a0, a1, a2, ones)
    o = (outp_ref[0, 0], outp_ref[0, 1], outp_ref[0, 2])
    wa = tuple(x * wmean for x in a)
    wyh, wyl = wyh_ref[...], wyl_ref[...]
    wxth, wxtl = wxth_ref[...], wxtl_ref[...]

    v = ([wa[i1] * a[i2] for (i1, i2) in _S_PAIRS]
         + [o[k] * wa[i1] for (k, i1) in _T_PAIRS])

    def dot3(p):
        ph, plo = _split_bf16(p)
        return (jnp.dot(wyh, ph, preferred_element_type=f32)
                + jnp.dot(wyh, plo, preferred_element_type=f32)
                + jnp.dot(wyl, ph, preferred_element_type=f32))

    def dot3r(q):
        qh, ql = _split_bf16(q)
        return (jnp.dot(qh, wxth, preferred_element_type=f32)
                + jnp.dot(qh, wxtl, preferred_element_type=f32)
                + jnp.dot(ql, wxth, preferred_element_type=f32))

    def zeta_step(zeta, _):
        uz = jnp.where(z0 == zeta, 1.0 - wz, 0.0) + jnp.where(z1 == zeta, wz, 0.0)
        for j in range(_NJ):
            g_ref[zeta, j] = dot3r(dot3(v[j] * uz))
        return 0

    jax.lax.fori_loop(0, _GD, zeta_step, 0, unroll=4)


def _inv4_sym(m):
    s0 = m[0, 0] * m[1, 1] - m[1, 0] * m[0, 1]
    s1 = m[0, 0] * m[1, 2] - m[1, 0] * m[0, 2]
    s2 = m[0, 0] * m[1, 3] - m[1, 0] * m[0, 3]
    s3 = m[0, 1] * m[1, 2] - m[1, 1] * m[0, 2]
    s4 = m[0, 1] * m[1, 3] - m[1, 1] * m[0, 3]
    s5 = m[0, 2] * m[1, 3] - m[1, 2] * m[0, 3]
    c5 = m[2, 2] * m[3, 3] - m[3, 2] * m[2, 3]
    c4 = m[2, 1] * m[3, 3] - m[3, 1] * m[2, 3]
    c3 = m[2, 1] * m[3, 2] - m[3, 1] * m[2, 2]
    c2 = m[2, 0] * m[3, 3] - m[3, 0] * m[2, 3]
    c1 = m[2, 0] * m[3, 2] - m[3, 0] * m[2, 2]
    c0 = m[2, 0] * m[3, 1] - m[3, 0] * m[2, 1]
    det = s0 * c5 - s1 * c4 + s2 * c3 + s3 * c2 - s4 * c1 + s5 * c0
    rdet = 1.0 / det
    inv = {}
    inv[0, 0] = (m[1, 1] * c5 - m[1, 2] * c4 + m[1, 3] * c3) * rdet
    inv[0, 1] = (-m[0, 1] * c5 + m[0, 2] * c4 - m[0, 3] * c3) * rdet
    inv[0, 2] = (m[3, 1] * s5 - m[3, 2] * s4 + m[3, 3] * s3) * rdet
    inv[0, 3] = (-m[2, 1] * s5 + m[2, 2] * s4 - m[2, 3] * s3) * rdet
    inv[1, 0] = (-m[1, 0] * c5 + m[1, 2] * c2 - m[1, 3] * c1) * rdet
    inv[1, 1] = (m[0, 0] * c5 - m[0, 2] * c2 + m[0, 3] * c1) * rdet
    inv[1, 2] = (-m[3, 0] * s5 + m[3, 2] * s2 - m[3, 3] * s1) * rdet
    inv[1, 3] = (m[2, 0] * s5 - m[2, 2] * s2 + m[2, 3] * s1) * rdet
    inv[2, 0] = (m[1, 0] * c4 - m[1, 1] * c2 + m[1, 3] * c0) * rdet
    inv[2, 1] = (-m[0, 0] * c4 + m[0, 1] * c2 - m[0, 3] * c0) * rdet
    inv[2, 2] = (m[3, 0] * s4 - m[3, 1] * s2 + m[3, 3] * s0) * rdet
    inv[2, 3] = (-m[2, 0] * s4 + m[2, 1] * s2 - m[2, 3] * s0) * rdet
    inv[3, 0] = (-m[1, 0] * c3 + m[1, 1] * c1 - m[1, 2] * c0) * rdet
    inv[3, 1] = (m[0, 0] * c3 - m[0, 1] * c1 + m[0, 2] * c0) * rdet
    inv[3, 2] = (-m[3, 0] * s3 + m[3, 1] * s1 - m[3, 2] * s0) * rdet
    inv[3, 3] = (m[2, 0] * s3 - m[2, 1] * s1 + m[2, 2] * s0) * rdet
    return inv


def _solve_body(g_ref, out_ref):
    S = {}
    for j, (i1, i2) in enumerate(_S_PAIRS):
        S[i1, i2] = g_ref[:, j]
        S[i2, i1] = S[i1, i2]
    T = {}
    for j, (k, i1) in enumerate(_T_PAIRS):
        T[k, i1] = g_ref[:, len(_S_PAIRS) + j]

    counts = S[3, 3]
    wl = _REG_LAMBDA * (counts + 1.0)

    gcs = jnp.sum(counts)
    wlg = _REG_LAMBDA * (gcs + 1.0)
    gain_g = [jnp.sum(T[k, 3]) / (jnp.sum(S[k, 3]) + wlg) for k in range(3)]
    zero_mask = counts == 0.0
    mixed = [jnp.where(zero_mask, gain_g[k], T[k, 3] / (S[k, 3] + wl))
             for k in range(3)]

    Sr = {}
    for i in range(4):
        for j in range(4):
            Sr[i, j] = S[i, j] + wl if i == j else S[i, j]
    Tr = {}
    for k in range(3):
        for i in range(4):
            Tr[k, i] = T[k, i] + wl * mixed[k] if i == k else T[k, i]

    amax = Sr[0, 0]
    for i in range(4):
        for j in range(4):
            amax = jnp.maximum(amax, jnp.abs(Sr[i, j]))
    scale = 1.0 / amax
    Sn = {k: v * scale for k, v in Sr.items()}
    inv = _inv4_sym(Sn)
    for k in range(3):
        for i in range(4):
            acc = Tr[k, 0] * inv[0, i]
            for q in range(1, 4):
                acc = acc + Tr[k, q] * inv[q, i]
            out_ref[k, i] = acc * scale


@jax.jit
def kernel(input_image, guide_image, output_image, weight_image):
    B, C, H, W = input_image.shape
    dtype = input_image.dtype
    wy = jnp.asarray(_spatial_weights(_GH, H))
    wxt = jnp.asarray(_spatial_weights(_GW, W).T)
    wyh, wyl = _split_bf16(wy)
    wxth, wxtl = _split_bf16(wxt)

    g = pl.pallas_call(
        _splat_body,
        out_shape=jax.ShapeDtypeStruct((_GD, _NJ, _GH, _GW), dtype),
    )(input_image, guide_image, output_image, weight_image,
      wyh, wyl, wxth, wxtl)

    gamma = pl.pallas_call(
        _solve_body,
        out_shape=jax.ShapeDtypeStruct((3, 4, _GD, _GH, _GW), dtype),
    )(g)

    return jnp.transpose(gamma, (3, 4, 2, 0, 1))[None]


# speedup vs baseline: 2272.0714x; 1.0822x over previous
import functools
import numpy as np

import jax
import jax.numpy as jnp
from jax.experimental import pallas as pl
from jax.experimental.pallas import tpu as pltpu

_GH = 24
_GW = 24
_GD = 16
_REG_LAMBDA = 1e-7

_S_PAIRS = [(i, j) for i in range(4) for j in range(i, 4)]
_T_PAIRS = [(k, i) for k in range(3) for i in range(4)]
_NJ = len(_S_PAIRS) + len(_T_PAIRS)


def _spatial_weights(g, n):
    pos = (np.arange(n, dtype=np.float64) + 0.5) * (g - 1) / n
    i0 = np.clip(np.floor(pos).astype(np.int64), 0, g - 1)
    i1 = np.minimum(i0 + 1, g - 1)
    w = (pos - i0).astype(np.float32)
    m = np.zeros((g, n), dtype=np.float32)
    m[i0, np.arange(n)] += 1.0 - w
    m[i1, np.arange(n)] += w
    return m


def _split_bf16(x):
    hi = x.astype(jnp.bfloat16)
    lo = (x - hi.astype(jnp.float32)).astype(jnp.bfloat16)
    return hi, lo


def _fused_body(inp_ref, guide_ref, outp_ref, wgt_ref, wyh_ref, wyl_ref,
                wxth_ref, wxtl_ref, out_ref, ph_s, pl_s, g_s):
    f32 = jnp.float32
    W = inp_ref.shape[3]
    a0 = inp_ref[0, 0]
    a1 = inp_ref[0, 1]
    a2 = inp_ref[0, 2]
    ones = jnp.ones_like(a0)
    wmean = (wgt_ref[0, 0] + wgt_ref[0, 1] + wgt_ref[0, 2]) * (1.0 / 3.0)
    gz = guide_ref[0, 0] * (_GD - 1)
    z0 = jnp.clip(jnp.floor(gz).astype(jnp.int32), 0, _GD - 1)
    z1 = jnp.minimum(z0 + 1, _GD - 1)
    wz = gz - z0.astype(f32)
    a = (a0, a1, a2, ones)
    o = (outp_ref[0, 0], outp_ref[0, 1], outp_ref[0, 2])
    wa = tuple(x * wmean for x in a)
    wyh, wyl = wyh_ref[...], wyl_ref[...]
    wxth, wxtl = wxth_ref[...], wxtl_ref[...]

    v = ([wa[i1] * a[i2] for (i1, i2) in _S_PAIRS]
         + [o[k] * wa[i1] for (k, i1) in _T_PAIRS])

    def dot3r(q):
        qh, ql = _split_bf16(q)
        return (jnp.dot(qh, wxth, preferred_element_type=f32)
                + jnp.dot(qh, wxtl, preferred_element_type=f32)
                + jnp.dot(ql, wxth, preferred_element_type=f32))

    def zeta_step(zeta, _):
        uz = jnp.where(z0 == zeta, 1.0 - wz, 0.0) + jnp.where(z1 == zeta, wz, 0.0)
        for j in range(_NJ):
            p = v[j] * uz
            ph, plo = _split_bf16(p)
            ph_s[:, j * W:(j + 1) * W] = ph
            pl_s[:, j * W:(j + 1) * W] = plo
        phv = ph_s[...]
        plv = pl_s[...]
        g1 = (jnp.dot(wyh, phv, preferred_element_type=f32)
              + jnp.dot(wyh, plv, preferred_element_type=f32)
              + jnp.dot(wyl, phv, preferred_element_type=f32))
        for j in range(_NJ):
            g_s[zeta, j] = dot3r(g1[:, j * W:(j + 1) * W])
        return 0

    jax.lax.fori_loop(0, _GD, zeta_step, 0, unroll=False)

    _solve_from([g_s[:, j] for j in range(_NJ)], out_ref)


def _inv4_sym(m):
    s0 = m[0, 0] * m[1, 1] - m[1, 0] * m[0, 1]
    s1 = m[0, 0] * m[1, 2] - m[1, 0] * m[0, 2]
    s2 = m[0, 0] * m[1, 3] - m[1, 0] * m[0, 3]
    s3 = m[0, 1] * m[1, 2] - m[1, 1] * m[0, 2]
    s4 = m[0, 1] * m[1, 3] - m[1, 1] * m[0, 3]
    s5 = m[0, 2] * m[1, 3] - m[1, 2] * m[0, 3]
    c5 = m[2, 2] * m[3, 3] - m[3, 2] * m[2, 3]
    c4 = m[2, 1] * m[3, 3] - m[3, 1] * m[2, 3]
    c3 = m[2, 1] * m[3, 2] - m[3, 1] * m[2, 2]
    c2 = m[2, 0] * m[3, 3] - m[3, 0] * m[2, 3]
    c1 = m[2, 0] * m[3, 2] - m[3, 0] * m[2, 2]
    c0 = m[2, 0] * m[3, 1] - m[3, 0] * m[2, 1]
    det = s0 * c5 - s1 * c4 + s2 * c3 + s3 * c2 - s4 * c1 + s5 * c0
    rdet = 1.0 / det
    inv = {}
    inv[0, 0] = (m[1, 1] * c5 - m[1, 2] * c4 + m[1, 3] * c3) * rdet
    inv[0, 1] = (-m[0, 1] * c5 + m[0, 2] * c4 - m[0, 3] * c3) * rdet
    inv[0, 2] = (m[3, 1] * s5 - m[3, 2] * s4 + m[3, 3] * s3) * rdet
    inv[0, 3] = (-m[2, 1] * s5 + m[2, 2] * s4 - m[2, 3] * s3) * rdet
    inv[1, 0] = (-m[1, 0] * c5 + m[1, 2] * c2 - m[1, 3] * c1) * rdet
    inv[1, 1] = (m[0, 0] * c5 - m[0, 2] * c2 + m[0, 3] * c1) * rdet
    inv[1, 2] = (-m[3, 0] * s5 + m[3, 2] * s2 - m[3, 3] * s1) * rdet
    inv[1, 3] = (m[2, 0] * s5 - m[2, 2] * s2 + m[2, 3] * s1) * rdet
    inv[2, 0] = (m[1, 0] * c4 - m[1, 1] * c2 + m[1, 3] * c0) * rdet
    inv[2, 1] = (-m[0, 0] * c4 + m[0, 1] * c2 - m[0, 3] * c0) * rdet
    inv[2, 2] = (m[3, 0] * s4 - m[3, 1] * s2 + m[3, 3] * s0) * rdet
    inv[2, 3] = (-m[2, 0] * s4 + m[2, 1] * s2 - m[2, 3] * s0) * rdet
    inv[3, 0] = (-m[1, 0] * c3 + m[1, 1] * c1 - m[1, 2] * c0) * rdet
    inv[3, 1] = (m[0, 0] * c3 - m[0, 1] * c1 + m[0, 2] * c0) * rdet
    inv[3, 2] = (-m[3, 0] * s3 + m[3, 1] * s1 - m[3, 2] * s0) * rdet
    inv[3, 3] = (m[2, 0] * s3 - m[2, 1] * s1 + m[2, 2] * s0) * rdet
    return inv


def _solve_from(g_arrs, out_ref):
    S = {}
    for j, (i1, i2) in enumerate(_S_PAIRS):
        S[i1, i2] = g_arrs[j]
        S[i2, i1] = S[i1, i2]
    T = {}
    for j, (k, i1) in enumerate(_T_PAIRS):
        T[k, i1] = g_arrs[len(_S_PAIRS) + j]

    counts = S[3, 3]
    wl = _REG_LAMBDA * (counts + 1.0)

    gcs = jnp.sum(counts)
    wlg = _REG_LAMBDA * (gcs + 1.0)
    gain_g = [jnp.sum(T[k, 3]) / (jnp.sum(S[k, 3]) + wlg) for k in range(3)]
    zero_mask = counts == 0.0
    mixed = [jnp.where(zero_mask, gain_g[k], T[k, 3] / (S[k, 3] + wl))
             for k in range(3)]

    Sr = {}
    for i in range(4):
        for j in range(4):
            Sr[i, j] = S[i, j] + wl if i == j else S[i, j]
    Tr = {}
    for k in range(3):
        for i in range(4):
            Tr[k, i] = T[k, i] + wl * mixed[k] if i == k else T[k, i]

    amax = Sr[0, 0]
    for i in range(4):
        for j in range(4):
            amax = jnp.maximum(amax, jnp.abs(Sr[i, j]))
    scale = 1.0 / amax
    Sn = {k: v * scale for k, v in Sr.items()}
    inv = _inv4_sym(Sn)
    for k in range(3):
        for i in range(4):
            acc = Tr[k, 0] * inv[0, i]
            for q in range(1, 4):
                acc = acc + Tr[k, q] * inv[q, i]
            out_ref[k, i] = acc * scale


@jax.jit
def kernel(input_image, guide_image, output_image, weight_image):
    B, C, H, W = input_image.shape
    dtype = input_image.dtype
    wy = jnp.asarray(_spatial_weights(_GH, H))
    wxt = jnp.asarray(_spatial_weights(_GW, W).T)
    wyh, wyl = _split_bf16(wy)
    wxth, wxtl = _split_bf16(wxt)

    gamma = pl.pallas_call(
        _fused_body,
        out_shape=jax.ShapeDtypeStruct((3, 4, _GD, _GH, _GW), dtype),
        scratch_shapes=[
            pltpu.VMEM((H, _NJ * W), jnp.bfloat16),
            pltpu.VMEM((H, _NJ * W), jnp.bfloat16),
            pltpu.VMEM((_GD, _NJ, _GH, _GW), dtype),
        ],
    )(input_image, guide_image, output_image, weight_image,
      wyh, wyl, wxth, wxtl)

    return jnp.transpose(gamma, (3, 4, 2, 0, 1))[None]


# speedup vs baseline: 2908.2277x; 1.2800x over previous
import functools
import numpy as np

import jax
import jax.numpy as jnp
from jax.experimental import pallas as pl
from jax.experimental.pallas import tpu as pltpu

_GH = 24
_GW = 24
_GD = 16
_REG_LAMBDA = 1e-7

_S_PAIRS = [(i, j) for i in range(4) for j in range(i, 4)]
_T_PAIRS = [(k, i) for k in range(3) for i in range(4)]
_NJ = len(_S_PAIRS) + len(_T_PAIRS)


def _spatial_weights(g, n):
    pos = (np.arange(n, dtype=np.float64) + 0.5) * (g - 1) / n
    i0 = np.clip(np.floor(pos).astype(np.int64), 0, g - 1)
    i1 = np.minimum(i0 + 1, g - 1)
    w = (pos - i0).astype(np.float32)
    m = np.zeros((g, n), dtype=np.float32)
    m[i0, np.arange(n)] += 1.0 - w
    m[i1, np.arange(n)] += w
    return m


def _split_bf16(x):
    hi = x.astype(jnp.bfloat16)
    lo = (x - hi.astype(jnp.float32)).astype(jnp.bfloat16)
    return hi, lo


def _fused_body(inp_ref, guide_ref, outp_ref, wgt_ref, wyh_ref, wyl_ref,
                wxth_ref, wxtl_ref, out_ref, ph_s, pl_s, g1_s):
    f32 = jnp.float32
    W = inp_ref.shape[3]
    a0 = inp_ref[0, 0]
    a1 = inp_ref[0, 1]
    a2 = inp_ref[0, 2]
    ones = jnp.ones_like(a0)
    wmean = (wgt_ref[0, 0] + wgt_ref[0, 1] + wgt_ref[0, 2]) * (1.0 / 3.0)
    gz = guide_ref[0, 0] * (_GD - 1)
    z0 = jnp.clip(jnp.floor(gz).astype(jnp.int32), 0, _GD - 1)
    z1 = jnp.minimum(z0 + 1, _GD - 1)
    wz = gz - z0.astype(f32)
    a = (a0, a1, a2, ones)
    o = (outp_ref[0, 0], outp_ref[0, 1], outp_ref[0, 2])
    wa = tuple(x * wmean for x in a)
    wyh, wyl = wyh_ref[...], wyl_ref[...]
    wxth, wxtl = wxth_ref[...], wxtl_ref[...]

    v = ([wa[i1] * a[i2] for (i1, i2) in _S_PAIRS]
         + [o[k] * wa[i1] for (k, i1) in _T_PAIRS])

    def zeta_step(zeta, _):
        uz = jnp.where(z0 == zeta, 1.0 - wz, 0.0) + jnp.where(z1 == zeta, wz, 0.0)
        for j in range(_NJ):
            p = v[j] * uz
            ph, plo = _split_bf16(p)
            ph_s[:, j * W:(j + 1) * W] = ph
            pl_s[:, j * W:(j + 1) * W] = plo
        phv = ph_s[...]
        plv = pl_s[...]
        g1 = (jnp.dot(wyh, phv, preferred_element_type=f32)
              + jnp.dot(wyh, plv, preferred_element_type=f32)
              + jnp.dot(wyl, phv, preferred_element_type=f32))
        g1_s[pl.ds(zeta * _GH, _GH), :] = g1
        return 0

    jax.lax.fori_loop(0, _GD, zeta_step, 0, unroll=False)

    g_arrs = []
    for j in range(_NJ):
        q = g1_s[:, j * W:(j + 1) * W]
        qh, ql = _split_bf16(q)
        r = (jnp.dot(qh, wxth, preferred_element_type=f32)
             + jnp.dot(qh, wxtl, preferred_element_type=f32)
             + jnp.dot(ql, wxth, preferred_element_type=f32))
        g_arrs.append(r.reshape(_GD, _GH, _GW))

    _solve_from(g_arrs, out_ref)


def _inv4_sym(m):
    s0 = m[0, 0] * m[1, 1] - m[1, 0] * m[0, 1]
    s1 = m[0, 0] * m[1, 2] - m[1, 0] * m[0, 2]
    s2 = m[0, 0] * m[1, 3] - m[1, 0] * m[0, 3]
    s3 = m[0, 1] * m[1, 2] - m[1, 1] * m[0, 2]
    s4 = m[0, 1] * m[1, 3] - m[1, 1] * m[0, 3]
    s5 = m[0, 2] * m[1, 3] - m[1, 2] * m[0, 3]
    c5 = m[2, 2] * m[3, 3] - m[3, 2] * m[2, 3]
    c4 = m[2, 1] * m[3, 3] - m[3, 1] * m[2, 3]
    c3 = m[2, 1] * m[3, 2] - m[3, 1] * m[2, 2]
    c2 = m[2, 0] * m[3, 3] - m[3, 0] * m[2, 3]
    c1 = m[2, 0] * m[3, 2] - m[3, 0] * m[2, 2]
    c0 = m[2, 0] * m[3, 1] - m[3, 0] * m[2, 1]
    det = s0 * c5 - s1 * c4 + s2 * c3 + s3 * c2 - s4 * c1 + s5 * c0
    rdet = 1.0 / det
    inv = {}
    inv[0, 0] = (m[1, 1] * c5 - m[1, 2] * c4 + m[1, 3] * c3) * rdet
    inv[0, 1] = (-m[0, 1] * c5 + m[0, 2] * c4 - m[0, 3] * c3) * rdet
    inv[0, 2] = (m[3, 1] * s5 - m[3, 2] * s4 + m[3, 3] * s3) * rdet
    inv[0, 3] = (-m[2, 1] * s5 + m[2, 2] * s4 - m[2, 3] * s3) * rdet
    inv[1, 0] = (-m[1, 0] * c5 + m[1, 2] * c2 - m[1, 3] * c1) * rdet
    inv[1, 1] = (m[0, 0] * c5 - m[0, 2] * c2 + m[0, 3] * c1) * rdet
    inv[1, 2] = (-m[3, 0] * s5 + m[3, 2] * s2 - m[3, 3] * s1) * rdet
    inv[1, 3] = (m[2, 0] * s5 - m[2, 2] * s2 + m[2, 3] * s1) * rdet
    inv[2, 0] = (m[1, 0] * c4 - m[1, 1] * c2 + m[1, 3] * c0) * rdet
    inv[2, 1] = (-m[0, 0] * c4 + m[0, 1] * c2 - m[0, 3] * c0) * rdet
    inv[2, 2] = (m[3, 0] * s4 - m[3, 1] * s2 + m[3, 3] * s0) * rdet
    inv[2, 3] = (-m[2, 0] * s4 + m[2, 1] * s2 - m[2, 3] * s0) * rdet
    inv[3, 0] = (-m[1, 0] * c3 + m[1, 1] * c1 - m[1, 2] * c0) * rdet
    inv[3, 1] = (m[0, 0] * c3 - m[0, 1] * c1 + m[0, 2] * c0) * rdet
    inv[3, 2] = (-m[3, 0] * s3 + m[3, 1] * s1 - m[3, 2] * s0) * rdet
    inv[3, 3] = (m[2, 0] * s3 - m[2, 1] * s1 + m[2, 2] * s0) * rdet
    return inv


def _solve_from(g_arrs, out_ref):
    S = {}
    for j, (i1, i2) in enumerate(_S_PAIRS):
        S[i1, i2] = g_arrs[j]
        S[i2, i1] = S[i1, i2]
    T = {}
    for j, (k, i1) in enumerate(_T_PAIRS):
        T[k, i1] = g_arrs[len(_S_PAIRS) + j]

    counts = S[3, 3]
    wl = _REG_LAMBDA * (counts + 1.0)

    gcs = jnp.sum(counts)
    wlg = _REG_LAMBDA * (gcs + 1.0)
    gain_g = [jnp.sum(T[k, 3]) / (jnp.sum(S[k, 3]) + wlg) for k in range(3)]
    zero_mask = counts == 0.0
    mixed = [jnp.where(zero_mask, gain_g[k], T[k, 3] / (S[k, 3] + wl))
             for k in range(3)]

    Sr = {}
    for i in range(4):
        for j in range(4):
            Sr[i, j] = S[i, j] + wl if i == j else S[i, j]
    Tr = {}
    for k in range(3):
        for i in range(4):
            Tr[k, i] = T[k, i] + wl * mixed[k] if i == k else T[k, i]

    amax = Sr[0, 0]
    for i in range(4):
        for j in range(4):
            amax = jnp.maximum(amax, jnp.abs(Sr[i, j]))
    scale = 1.0 / amax
    Sn = {k: v * scale for k, v in Sr.items()}
    inv = _inv4_sym(Sn)
    for k in range(3):
        for i in range(4):
            acc = Tr[k, 0] * inv[0, i]
            for q in range(1, 4):
                acc = acc + Tr[k, q] * inv[q, i]
            out_ref[k, i] = acc * scale


@jax.jit
def kernel(input_image, guide_image, output_image, weight_image):
    B, C, H, W = input_image.shape
    dtype = input_image.dtype
    wy = jnp.asarray(_spatial_weights(_GH, H))
    wxt = jnp.asarray(_spatial_weights(_GW, W).T)
    wyh, wyl = _split_bf16(wy)
    wxth, wxtl = _split_bf16(wxt)

    gamma = pl.pallas_call(
        _fused_body,
        out_shape=jax.ShapeDtypeStruct((3, 4, _GD, _GH, _GW), dtype),
        scratch_shapes=[
            pltpu.VMEM((H, _NJ * W), jnp.bfloat16),
            pltpu.VMEM((H, _NJ * W), jnp.bfloat16),
            pltpu.VMEM((_GD * _GH, _NJ * W), dtype),
        ],
    )(input_image, guide_image, output_image, weight_image,
      wyh, wyl, wxth, wxtl)

    return jnp.transpose(gamma, (3, 4, 2, 0, 1))[None]
